# baseline (device time: 178206 ns/iter reference)
import jax
import jax.numpy as jnp
from jax import lax
from jax.experimental import pallas as pl
from jax.experimental.pallas import tpu as pltpu

N_DEV = 4


def kernel(x, Wq, K_ext, V_ext, Wo):
    B, Sq, E = x.shape
    _, Skv, Hq, Dh = K_ext.shape
    BH = B * Hq
    QB = 64

    Wqh = Wq.reshape(E, Hq, Dh).transpose(1, 0, 2)
    Kh = K_ext.transpose(0, 2, 1, 3).reshape(BH, Skv, Dh)
    Vh = V_ext.transpose(0, 2, 1, 3).reshape(BH, Skv, Dh)

    def body(x_ref, wqh_ref, kh_ref, vh_ref, wo_ref, out_ref,
             comm_ctx, comm_l, ctx_send_sems, ctx_recv_sems,
             l_send_sems, l_recv_sems):
        my = lax.axis_index("i")
        left = lax.rem(my - 1 + N_DEV, N_DEV)
        right = lax.rem(my + 1, N_DEV)

        barrier_sem = pltpu.get_barrier_semaphore()
        for nbr in (left, right):
            pl.semaphore_signal(barrier_sem, inc=1, device_id=(nbr,),
                                device_id_type=pl.DeviceIdType.MESH)
        pl.semaphore_wait(barrier_sem, 2)

        qb = lax.broadcasted_iota(jnp.int32, (Sq, Skv), 0) // QB
        kb = my * (Skv // QB) + lax.broadcasted_iota(jnp.int32, (Sq, Skv), 1) // QB
        mask = (qb == kb) | (kb == 0) | ((qb + kb) % 3 == 0)

        for bh in range(BH):
            b, h = bh // Hq, bh % Hq
            q = jnp.dot(x_ref[b], wqh_ref[h],
                        preferred_element_type=jnp.float32)
            s = lax.dot_general(q, kh_ref[bh], (((1,), (1,)), ((), ())),
                                preferred_element_type=jnp.float32) * 0.125
            w = jnp.exp(jnp.where(mask, s, -1e9))
            comm_l[0, :, bh:bh + 1] = jnp.sum(w, axis=1, keepdims=True)
            comm_ctx[0, bh] = jnp.dot(w, vh_ref[bh],
                                      preferred_element_type=jnp.float32)

        for h in range(N_DEV - 1):
            rdma_ctx = pltpu.make_async_remote_copy(
                src_ref=comm_ctx.at[h], dst_ref=comm_ctx.at[h + 1],
                send_sem=ctx_send_sems.at[h], recv_sem=ctx_recv_sems.at[h],
                device_id=(right,), device_id_type=pl.DeviceIdType.MESH,
            )
            rdma_l = pltpu.make_async_remote_copy(
                src_ref=comm_l.at[h], dst_ref=comm_l.at[h + 1],
                send_sem=l_send_sems.at[h], recv_sem=l_recv_sems.at[h],
                device_id=(right,), device_id_type=pl.DeviceIdType.MESH,
            )
            rdma_ctx.start()
            rdma_l.start()
            rdma_ctx.wait()
            rdma_l.wait()

        l_sum = (comm_l[0] + comm_l[1]) + (comm_l[2] + comm_l[3])
        for b in range(B):
            acc = jnp.zeros((Sq, E), jnp.float32)
            for h in range(Hq):
                bh = b * Hq + h
                ctx = ((comm_ctx[0, bh] + comm_ctx[1, bh])
                       + (comm_ctx[2, bh] + comm_ctx[3, bh]))
                ctx = ctx / l_sum[:, bh:bh + 1]
                acc = acc + jnp.dot(ctx, wo_ref[pl.ds(h * Dh, Dh), :],
                                    preferred_element_type=jnp.float32)
            out_ref[b] = acc

    return pl.pallas_call(
        body,
        out_shape=jax.ShapeDtypeStruct((B, Sq, E), jnp.float32),
        in_specs=[pl.BlockSpec(memory_space=pltpu.VMEM)] * 5,
        out_specs=pl.BlockSpec(memory_space=pltpu.VMEM),
        scratch_shapes=[
            pltpu.VMEM((N_DEV, BH, Sq, Dh), jnp.float32),
            pltpu.VMEM((N_DEV, Sq, BH), jnp.float32),
            pltpu.SemaphoreType.DMA((N_DEV - 1,)),
            pltpu.SemaphoreType.DMA((N_DEV - 1,)),
            pltpu.SemaphoreType.DMA((N_DEV - 1,)),
            pltpu.SemaphoreType.DMA((N_DEV - 1,)),
        ],
        compiler_params=pltpu.CompilerParams(collective_id=0),
    )(x, Wqh, Kh, Vh, Wo)


# device time: 177931 ns/iter; 1.0015x vs baseline; 1.0015x over previous
import jax
import jax.numpy as jnp
from jax import lax
from jax.experimental import pallas as pl
from jax.experimental.pallas import tpu as pltpu

N_DEV = 4


def kernel(x, Wq, K_ext, V_ext, Wo):
    B, Sq, E = x.shape
    _, Skv, Hq, Dh = K_ext.shape
    BH = B * Hq
    QB = 64

    bf16 = jnp.bfloat16
    Wqh = Wq.reshape(E, Hq, Dh).transpose(1, 0, 2).astype(bf16)
    Kh = K_ext.transpose(0, 2, 1, 3).reshape(BH, Skv, Dh).astype(bf16)
    Vh = V_ext.transpose(0, 2, 1, 3).reshape(BH, Skv, Dh).astype(bf16)
    xb = x.astype(bf16)
    Wob = Wo.astype(bf16)

    def body(x_ref, wqh_ref, kh_ref, vh_ref, wo_ref, out_ref,
             comm_ctx, comm_l, ctx_send_sems, ctx_recv_sems,
             l_send_sems, l_recv_sems):
        my = lax.axis_index("i")
        left = lax.rem(my - 1 + N_DEV, N_DEV)
        right = lax.rem(my + 1, N_DEV)

        barrier_sem = pltpu.get_barrier_semaphore()
        for nbr in (left, right):
            pl.semaphore_signal(barrier_sem, inc=1, device_id=(nbr,),
                                device_id_type=pl.DeviceIdType.MESH)
        pl.semaphore_wait(barrier_sem, 2)

        qb = lax.broadcasted_iota(jnp.int32, (Sq, Skv), 0) // QB
        kb = my * (Skv // QB) + lax.broadcasted_iota(jnp.int32, (Sq, Skv), 1) // QB
        mask = (qb == kb) | (kb == 0) | ((qb + kb) % 3 == 0)

        for bh in range(BH):
            b, h = bh // Hq, bh % Hq
            q = jnp.dot(x_ref[b], wqh_ref[h],
                        preferred_element_type=jnp.float32)
            s = lax.dot_general(q.astype(jnp.bfloat16), kh_ref[bh],
                                (((1,), (1,)), ((), ())),
                                preferred_element_type=jnp.float32) * 0.125
            w = jnp.exp(jnp.where(mask, s, -1e9))
            comm_l[0, :, bh:bh + 1] = jnp.sum(w, axis=1, keepdims=True)
            comm_ctx[0, bh] = jnp.dot(w.astype(jnp.bfloat16), vh_ref[bh],
                                      preferred_element_type=jnp.float32)

        for h in range(N_DEV - 1):
            rdma_ctx = pltpu.make_async_remote_copy(
                src_ref=comm_ctx.at[h], dst_ref=comm_ctx.at[h + 1],
                send_sem=ctx_send_sems.at[h], recv_sem=ctx_recv_sems.at[h],
                device_id=(right,), device_id_type=pl.DeviceIdType.MESH,
            )
            rdma_l = pltpu.make_async_remote_copy(
                src_ref=comm_l.at[h], dst_ref=comm_l.at[h + 1],
                send_sem=l_send_sems.at[h], recv_sem=l_recv_sems.at[h],
                device_id=(right,), device_id_type=pl.DeviceIdType.MESH,
            )
            rdma_ctx.start()
            rdma_l.start()
            rdma_ctx.wait()
            rdma_l.wait()

        l_sum = (comm_l[0] + comm_l[1]) + (comm_l[2] + comm_l[3])
        for b in range(B):
            acc = jnp.zeros((Sq, E), jnp.float32)
            for h in range(Hq):
                bh = b * Hq + h
                ctx = ((comm_ctx[0, bh] + comm_ctx[1, bh])
                       + (comm_ctx[2, bh] + comm_ctx[3, bh]))
                ctx = (ctx / l_sum[:, bh:bh + 1]).astype(jnp.bfloat16)
                acc = acc + jnp.dot(ctx, wo_ref[pl.ds(h * Dh, Dh), :],
                                    preferred_element_type=jnp.float32)
            out_ref[b] = acc

    return pl.pallas_call(
        body,
        out_shape=jax.ShapeDtypeStruct((B, Sq, E), jnp.float32),
        in_specs=[pl.BlockSpec(memory_space=pltpu.VMEM)] * 5,
        out_specs=pl.BlockSpec(memory_space=pltpu.VMEM),
        scratch_shapes=[
            pltpu.VMEM((N_DEV, BH, Sq, Dh), jnp.float32),
            pltpu.VMEM((N_DEV, Sq, BH), jnp.float32),
            pltpu.SemaphoreType.DMA((N_DEV - 1,)),
            pltpu.SemaphoreType.DMA((N_DEV - 1,)),
            pltpu.SemaphoreType.DMA((N_DEV - 1,)),
            pltpu.SemaphoreType.DMA((N_DEV - 1,)),
        ],
        compiler_params=pltpu.CompilerParams(collective_id=0),
    )(xb, Wqh, Kh, Vh, Wob)


# device time: 53749 ns/iter; 3.3155x vs baseline; 3.3104x over previous
import jax
import jax.numpy as jnp
from jax import lax
from jax.experimental import pallas as pl
from jax.experimental.pallas import tpu as pltpu

N_DEV = 4


def kernel(x, Wq, K_ext, V_ext, Wo):
    B, Sq, E = x.shape
    _, Skv, Hq, Dh = K_ext.shape
    BH = B * Hq
    NP = BH // 2
    NR = NP // 2
    QB = 64

    bf16 = jnp.bfloat16
    Wqh = Wq.reshape(E, Hq, Dh).transpose(1, 0, 2).astype(bf16)
    Kh = K_ext.transpose(0, 2, 1, 3).reshape(BH, Skv, Dh).astype(bf16)
    Vh = V_ext.transpose(0, 2, 1, 3).reshape(BH, Skv, Dh).astype(bf16)
    xb = x.astype(bf16)
    Wob = Wo.astype(bf16)

    def body(x_ref, wqh_ref, kh_ref, vh_ref, wo_ref, out_ref,
             commR, commL, sendR, recvR, sendL, recvL):
        my = lax.axis_index("i")
        left = lax.rem(my - 1 + N_DEV, N_DEV)
        right = lax.rem(my + 1, N_DEV)

        barrier_sem = pltpu.get_barrier_semaphore()
        for nbr in (left, right):
            pl.semaphore_signal(barrier_sem, inc=1, device_id=(nbr,),
                                device_id_type=pl.DeviceIdType.MESH)
        pl.semaphore_wait(barrier_sem, 2)

        qb = lax.broadcasted_iota(jnp.int32, (Sq, Skv), 0) // QB
        kb = my * (Skv // QB) + lax.broadcasted_iota(jnp.int32, (Sq, Skv), 1) // QB
        mask = (qb == kb) | (kb == 0) | ((qb + kb) % 3 == 0)

        def partial(bh):
            b, h = bh // Hq, bh % Hq
            q = jnp.dot(x_ref[b], wqh_ref[h],
                        preferred_element_type=jnp.float32)
            s = lax.dot_general(q.astype(bf16), kh_ref[bh],
                                (((1,), (1,)), ((), ())),
                                preferred_element_type=jnp.float32) * 0.125
            w = jnp.exp(jnp.where(mask, s, -1e9))
            l = jnp.sum(w, axis=1, keepdims=True)
            ctx = jnp.dot(w.astype(bf16), vh_ref[bh],
                          preferred_element_type=jnp.float32)
            return ctx, l

        for p in range(NP):
            ctx0, l0 = partial(2 * p)
            ctx1, l1 = partial(2 * p + 1)
            pair = jnp.concatenate([ctx0, ctx1], axis=1).astype(bf16)
            comm, row = (commR, p) if p < NR else (commL, p - NR)
            comm[0, row] = pair
            comm[0, NR, :, 2 * (p % NR):2 * (p % NR) + 1] = l0.astype(bf16)
            comm[0, NR, :, 2 * (p % NR) + 1:2 * (p % NR) + 2] = l1.astype(bf16)

        for hop in range(N_DEV - 1):
            rR = pltpu.make_async_remote_copy(
                src_ref=commR.at[hop], dst_ref=commR.at[hop + 1],
                send_sem=sendR.at[hop], recv_sem=recvR.at[hop],
                device_id=(right,), device_id_type=pl.DeviceIdType.MESH,
            )
            rL = pltpu.make_async_remote_copy(
                src_ref=commL.at[hop], dst_ref=commL.at[hop + 1],
                send_sem=sendL.at[hop], recv_sem=recvL.at[hop],
                device_id=(left,), device_id_type=pl.DeviceIdType.MESH,
            )
            rR.start()
            rL.start()
            rR.wait()
            rL.wait()

        lRs = (commR[0, NR].astype(jnp.float32) + commR[1, NR].astype(jnp.float32)
               + commR[2, NR].astype(jnp.float32) + commR[3, NR].astype(jnp.float32))
        lLs = (commL[0, NR].astype(jnp.float32) + commL[1, NR].astype(jnp.float32)
               + commL[2, NR].astype(jnp.float32) + commL[3, NR].astype(jnp.float32))
        for b in range(B):
            acc = jnp.zeros((Sq, E), jnp.float32)
            for pb in range(Hq // 2):
                p = b * (Hq // 2) + pb
                comm, row = (commR, p) if p < NR else (commL, p - NR)
                ls = lRs if p < NR else lLs
                pair = (comm[0, row].astype(jnp.float32)
                        + comm[1, row].astype(jnp.float32)
                        + comm[2, row].astype(jnp.float32)
                        + comm[3, row].astype(jnp.float32))
                div = jnp.concatenate(
                    [jnp.broadcast_to(ls[:, 2 * (p % NR):2 * (p % NR) + 1], (Sq, Dh)),
                     jnp.broadcast_to(ls[:, 2 * (p % NR) + 1:2 * (p % NR) + 2], (Sq, Dh))],
                    axis=1)
                ctxn = (pair / div).astype(bf16)
                acc = acc + jnp.dot(ctxn, wo_ref[pl.ds(pb * 2 * Dh, 2 * Dh), :],
                                    preferred_element_type=jnp.float32)
            out_ref[b] = acc

    return pl.pallas_call(
        body,
        out_shape=jax.ShapeDtypeStruct((B, Sq, E), jnp.float32),
        in_specs=[pl.BlockSpec(memory_space=pltpu.VMEM)] * 5,
        out_specs=pl.BlockSpec(memory_space=pltpu.VMEM),
        scratch_shapes=[
            pltpu.VMEM((N_DEV, NR + 1, Sq, 2 * Dh), bf16),
            pltpu.VMEM((N_DEV, NR + 1, Sq, 2 * Dh), bf16),
            pltpu.SemaphoreType.DMA((N_DEV - 1,)),
            pltpu.SemaphoreType.DMA((N_DEV - 1,)),
            pltpu.SemaphoreType.DMA((N_DEV - 1,)),
            pltpu.SemaphoreType.DMA((N_DEV - 1,)),
        ],
        compiler_params=pltpu.CompilerParams(collective_id=0),
    )(xb, Wqh, Kh, Vh, Wob)


# device time: 49875 ns/iter; 3.5731x vs baseline; 1.0777x over previous
import jax
import jax.numpy as jnp
from jax import lax
from jax.experimental import pallas as pl
from jax.experimental.pallas import tpu as pltpu

N_DEV = 4


def kernel(x, Wq, K_ext, V_ext, Wo):
    B, Sq, E = x.shape
    _, Skv, Hq, Dh = K_ext.shape
    BH = B * Hq
    NP = BH // 2
    NR = NP // 2
    QB = 64

    bf16 = jnp.bfloat16
    Wqp = Wq.reshape(E, NR, 2 * Dh).transpose(1, 0, 2).astype(bf16)
    Kh = K_ext.transpose(0, 2, 1, 3).reshape(BH, Skv, Dh).astype(bf16)
    Vh = V_ext.transpose(0, 2, 1, 3).reshape(BH, Skv, Dh).astype(bf16)
    xb = x.astype(bf16)
    Wob = Wo.astype(bf16)

    def body(x_ref, wqp_ref, kh_ref, vh_ref, wo_ref, out_ref,
             commR, commL, sendR, recvR, sendL, recvL):
        f32 = jnp.float32
        my = lax.axis_index("i")
        left = lax.rem(my - 1 + N_DEV, N_DEV)
        right = lax.rem(my + 1, N_DEV)

        barrier_sem = pltpu.get_barrier_semaphore()
        for nbr in (left, right):
            pl.semaphore_signal(barrier_sem, inc=1, device_id=(nbr,),
                                device_id_type=pl.DeviceIdType.MESH)
        pl.semaphore_wait(barrier_sem, 2)

        qb = lax.broadcasted_iota(jnp.int32, (Sq, Skv), 0) // QB
        kb = my * (Skv // QB) + lax.broadcasted_iota(jnp.int32, (Sq, Skv), 1) // QB
        mask = (qb == kb) | (kb == 0) | ((qb + kb) % 3 == 0)

        def partial_pair(p):
            b, pb = p // NR, p % NR
            qp = jnp.dot(x_ref[b], wqp_ref[pb],
                         preferred_element_type=f32)
            ctxs, ls = [], []
            for k in (0, 1):
                bh = 2 * p + k
                q = qp[:, k * Dh:(k + 1) * Dh].astype(bf16)
                s = lax.dot_general(q, kh_ref[bh], (((1,), (1,)), ((), ())),
                                    preferred_element_type=f32) * 0.125
                w = jnp.exp(jnp.where(mask, s, -1e9))
                ls.append(jnp.sum(w, axis=1, keepdims=True))
                ctxs.append(jnp.dot(w.astype(bf16), vh_ref[bh],
                                    preferred_element_type=f32))
            return jnp.concatenate(ctxs, axis=1), ls[0], ls[1]

        def mk(comm, hop, sends, recvs, tgt):
            return pltpu.make_async_remote_copy(
                src_ref=comm.at[hop], dst_ref=comm.at[hop + 1],
                send_sem=sends.at[hop], recv_sem=recvs.at[hop],
                device_id=(tgt,), device_id_type=pl.DeviceIdType.MESH)

        accs = [None] * (NP + 2)
        for p in range(NR):
            pair, l0, l1 = partial_pair(p)
            commR[0, p] = pair.astype(bf16)
            commR[0, NR, :, 2 * p:2 * p + 1] = l0.astype(bf16)
            commR[0, NR, :, 2 * p + 1:2 * p + 2] = l1.astype(bf16)
            accs[p] = pair
        dR = [None] * (N_DEV - 1)
        dL = [None] * (N_DEV - 1)
        dR[0] = mk(commR, 0, sendR, recvR, right)
        dR[0].start()

        for p in range(NR, NP):
            pair, l0, l1 = partial_pair(p)
            pb = p - NR
            commL[0, pb] = pair.astype(bf16)
            commL[0, NR, :, 2 * pb:2 * pb + 1] = l0.astype(bf16)
            commL[0, NR, :, 2 * pb + 1:2 * pb + 2] = l1.astype(bf16)
            accs[p] = pair
        dL[0] = mk(commL, 0, sendL, recvL, left)
        dL[0].start()
        accs[NP] = commR[0, NR].astype(f32)
        accs[NP + 1] = commL[0, NR].astype(f32)

        for hop in range(N_DEV - 1):
            dR[hop].wait_recv()
            if hop + 1 < N_DEV - 1:
                dR[hop + 1] = mk(commR, hop + 1, sendR, recvR, right)
                dR[hop + 1].start()
            for r in range(NR):
                accs[r] = accs[r] + commR[hop + 1, r].astype(f32)
            accs[NP] = accs[NP] + commR[hop + 1, NR].astype(f32)

            dL[hop].wait_recv()
            if hop + 1 < N_DEV - 1:
                dL[hop + 1] = mk(commL, hop + 1, sendL, recvL, left)
                dL[hop + 1].start()
            for r in range(NR):
                accs[NR + r] = accs[NR + r] + commL[hop + 1, r].astype(f32)
            accs[NP + 1] = accs[NP + 1] + commL[hop + 1, NR].astype(f32)

        for b in range(B):
            acc = jnp.zeros((Sq, E), f32)
            lrow = accs[NP + b]
            for pb in range(NR):
                pair = accs[b * NR + pb]
                div = jnp.concatenate(
                    [jnp.broadcast_to(lrow[:, 2 * pb:2 * pb + 1], (Sq, Dh)),
                     jnp.broadcast_to(lrow[:, 2 * pb + 1:2 * pb + 2], (Sq, Dh))],
                    axis=1)
                ctxn = (pair / div).astype(bf16)
                acc = acc + jnp.dot(ctxn, wo_ref[pl.ds(pb * 2 * Dh, 2 * Dh), :],
                                    preferred_element_type=f32)
            out_ref[b] = acc

        for hop in range(N_DEV - 1):
            dR[hop].wait_send()
            dL[hop].wait_send()

    return pl.pallas_call(
        body,
        out_shape=jax.ShapeDtypeStruct((B, Sq, E), jnp.float32),
        in_specs=[pl.BlockSpec(memory_space=pltpu.VMEM)] * 5,
        out_specs=pl.BlockSpec(memory_space=pltpu.VMEM),
        scratch_shapes=[
            pltpu.VMEM((N_DEV, NR + 1, Sq, 2 * Dh), bf16),
            pltpu.VMEM((N_DEV, NR + 1, Sq, 2 * Dh), bf16),
            pltpu.SemaphoreType.DMA((N_DEV - 1,)),
            pltpu.SemaphoreType.DMA((N_DEV - 1,)),
            pltpu.SemaphoreType.DMA((N_DEV - 1,)),
            pltpu.SemaphoreType.DMA((N_DEV - 1,)),
        ],
        compiler_params=pltpu.CompilerParams(collective_id=0),
    )(xb, Wqp, Kh, Vh, Wob)


# device time: 45958 ns/iter; 3.8776x vs baseline; 1.0852x over previous
import jax
import jax.numpy as jnp
from jax import lax
from jax.experimental import pallas as pl
from jax.experimental.pallas import tpu as pltpu

N_DEV = 4
NH = 2


def kernel(x, Wq, K_ext, V_ext, Wo):
    B, Sq, E = x.shape
    _, Skv, Hq, Dh = K_ext.shape
    BH = B * Hq
    NP = BH // 2
    NR = NP // 2
    QB = 64
    SH = Sq // NH

    bf16 = jnp.bfloat16
    Wqp = Wq.reshape(E, NR, 2 * Dh).transpose(1, 0, 2).astype(bf16)
    Kh = K_ext.transpose(0, 2, 1, 3).reshape(BH, Skv, Dh).astype(bf16)
    Vh = V_ext.transpose(0, 2, 1, 3).reshape(BH, Skv, Dh).astype(bf16)
    xb = x.astype(bf16)
    Wob = Wo.astype(bf16)

    def body(x_ref, wqp_ref, kh_ref, vh_ref, wo_ref, out_ref,
             commR, commL, sendR, recvR, sendL, recvL):
        f32 = jnp.float32
        my = lax.axis_index("i")
        left = lax.rem(my - 1 + N_DEV, N_DEV)
        right = lax.rem(my + 1, N_DEV)

        barrier_sem = pltpu.get_barrier_semaphore()
        for nbr in (left, right):
            pl.semaphore_signal(barrier_sem, inc=1, device_id=(nbr,),
                                device_id_type=pl.DeviceIdType.MESH)
        pl.semaphore_wait(barrier_sem, 2)

        qb = lax.broadcasted_iota(jnp.int32, (Sq, Skv), 0) // QB
        kb = my * (Skv // QB) + lax.broadcasted_iota(jnp.int32, (Sq, Skv), 1) // QB
        mask = (qb == kb) | (kb == 0) | ((qb + kb) % 3 == 0)

        def partial_half(p, half):
            b, pb = p // NR, p % NR
            qp = jnp.dot(x_ref[b, pl.ds(half * SH, SH), :], wqp_ref[pb],
                         preferred_element_type=f32)
            m = mask[half * SH:(half + 1) * SH]
            ctxs, ls = [], []
            for k in (0, 1):
                bh = 2 * p + k
                q = qp[:, k * Dh:(k + 1) * Dh].astype(bf16)
                s = lax.dot_general(q, kh_ref[bh], (((1,), (1,)), ((), ())),
                                    preferred_element_type=f32) * 0.125
                w = jnp.exp(jnp.where(m, s, -1e9))
                ls.append(jnp.sum(w, axis=1, keepdims=True))
                ctxs.append(jnp.dot(w.astype(bf16), vh_ref[bh],
                                    preferred_element_type=f32))
            return jnp.concatenate(ctxs, axis=1), ls[0], ls[1]

        def mk(comm, hop, half, sends, recvs, tgt):
            return pltpu.make_async_remote_copy(
                src_ref=comm.at[hop, half], dst_ref=comm.at[hop + 1, half],
                send_sem=sends.at[hop, half], recv_sem=recvs.at[hop, half],
                device_id=(tgt,), device_id_type=pl.DeviceIdType.MESH)

        accs = [[[None] * (NR + 1) for _ in range(NH)] for _ in range(2)]
        dR = [[None] * NH for _ in range(N_DEV - 1)]
        dL = [[None] * NH for _ in range(N_DEV - 1)]

        for half in range(NH):
            for p in range(NP):
                pair, l0, l1 = partial_half(p, half)
                d, pb = (0, p) if p < NR else (1, p - NR)
                comm = commR if d == 0 else commL
                comm[0, half, pb] = pair.astype(bf16)
                comm[0, half, NR, :, 2 * pb:2 * pb + 1] = l0.astype(bf16)
                comm[0, half, NR, :, 2 * pb + 1:2 * pb + 2] = l1.astype(bf16)
                accs[d][half][pb] = pair
            dR[0][half] = mk(commR, 0, half, sendR, recvR, right)
            dR[0][half].start()
            dL[0][half] = mk(commL, 0, half, sendL, recvL, left)
            dL[0][half].start()
            accs[0][half][NR] = commR[0, half, NR].astype(f32)
            accs[1][half][NR] = commL[0, half, NR].astype(f32)

        def finish(b, half):
            lrow = accs[b][half][NR]
            acc = jnp.zeros((SH, E), f32)
            for pb in range(NR):
                pair = accs[b][half][pb]
                div = jnp.concatenate(
                    [jnp.broadcast_to(lrow[:, 2 * pb:2 * pb + 1], (SH, Dh)),
                     jnp.broadcast_to(lrow[:, 2 * pb + 1:2 * pb + 2], (SH, Dh))],
                    axis=1)
                ctxn = (pair / div).astype(bf16)
                acc = acc + jnp.dot(ctxn, wo_ref[pl.ds(pb * 2 * Dh, 2 * Dh), :],
                                    preferred_element_type=f32)
            out_ref[b, pl.ds(half * SH, SH), :] = acc

        for hop in range(N_DEV - 1):
            for half in range(NH):
                for d, dd, comm in ((0, dR, commR), (1, dL, commL)):
                    dd[hop][half].wait_recv()
                    if hop + 1 < N_DEV - 1:
                        dd[hop + 1][half] = mk(
                            comm, hop + 1, half,
                            sendR if d == 0 else sendL,
                            recvR if d == 0 else recvL,
                            right if d == 0 else left)
                        dd[hop + 1][half].start()
                    for r in range(NR + 1):
                        accs[d][half][r] = (accs[d][half][r]
                                            + comm[hop + 1, half, r].astype(f32))
                if hop == N_DEV - 2:
                    finish(0, half)
                    finish(1, half)

        for hop in range(N_DEV - 1):
            for half in range(NH):
                dR[hop][half].wait_send()
                dL[hop][half].wait_send()

    return pl.pallas_call(
        body,
        out_shape=jax.ShapeDtypeStruct((B, Sq, E), jnp.float32),
        in_specs=[pl.BlockSpec(memory_space=pltpu.VMEM)] * 5,
        out_specs=pl.BlockSpec(memory_space=pltpu.VMEM),
        scratch_shapes=[
            pltpu.VMEM((N_DEV, NH, NR + 1, SH, 2 * Dh), bf16),
            pltpu.VMEM((N_DEV, NH, NR + 1, SH, 2 * Dh), bf16),
            pltpu.SemaphoreType.DMA((N_DEV - 1, NH)),
            pltpu.SemaphoreType.DMA((N_DEV - 1, NH)),
            pltpu.SemaphoreType.DMA((N_DEV - 1, NH)),
            pltpu.SemaphoreType.DMA((N_DEV - 1, NH)),
        ],
        compiler_params=pltpu.CompilerParams(collective_id=0),
    )(xb, Wqp, Kh, Vh, Wob)


# device time: 40713 ns/iter; 4.3771x vs baseline; 1.1288x over previous
import jax
import jax.numpy as jnp
from jax import lax
from jax.experimental import pallas as pl
from jax.experimental.pallas import tpu as pltpu

N_DEV = 4
NH = 2


def kernel(x, Wq, K_ext, V_ext, Wo):
    B, Sq, E = x.shape
    _, Skv, Hq, Dh = K_ext.shape
    BH = B * Hq
    NP = BH // 2
    NR = NP // 2
    QB = 64
    SH = Sq // NH

    bf16 = jnp.bfloat16
    Wqb = Wq.astype(bf16)
    Kh = K_ext.transpose(0, 2, 1, 3).reshape(BH, Skv, Dh).astype(bf16)
    Vh = V_ext.transpose(0, 2, 1, 3).reshape(BH, Skv, Dh).astype(bf16)
    xb = x.astype(bf16)
    Wob = Wo.astype(bf16)

    def body(x_ref, wq_ref, kh_ref, vh_ref, wo_ref, out_ref,
             commR, commL, sendR, recvR, sendL, recvL):
        f32 = jnp.float32
        my = lax.axis_index("i")
        left = lax.rem(my - 1 + N_DEV, N_DEV)
        right = lax.rem(my + 1, N_DEV)

        barrier_sem = pltpu.get_barrier_semaphore()
        for nbr in (left, right):
            pl.semaphore_signal(barrier_sem, inc=1, device_id=(nbr,),
                                device_id_type=pl.DeviceIdType.MESH)
        pl.semaphore_wait(barrier_sem, 2)

        qb = lax.broadcasted_iota(jnp.int32, (Sq, Skv), 0) // QB
        kb = my * (Skv // QB) + lax.broadcasted_iota(jnp.int32, (Sq, Skv), 1) // QB
        mask = (qb == kb) | (kb == 0) | ((qb + kb) % 3 == 0)

        qcache = {}

        def qfull(b, half):
            if (b, half) not in qcache:
                qcache[(b, half)] = jnp.dot(
                    x_ref[b, pl.ds(half * SH, SH), :], wq_ref[:, :],
                    preferred_element_type=f32)
            return qcache[(b, half)]

        def partial_half(p, half):
            b, pb = p // NR, p % NR
            qp = qfull(b, half)[:, pb * 2 * Dh:(pb + 1) * 2 * Dh]
            m = mask[half * SH:(half + 1) * SH]
            ctxs, ls = [], []
            for k in (0, 1):
                bh = 2 * p + k
                q = qp[:, k * Dh:(k + 1) * Dh].astype(bf16)
                s = lax.dot_general(q, kh_ref[bh], (((1,), (1,)), ((), ())),
                                    preferred_element_type=f32) * 0.125
                w = jnp.exp(jnp.where(m, s, -1e9))
                ls.append(jnp.sum(w, axis=1, keepdims=True))
                ctxs.append(jnp.dot(w.astype(bf16), vh_ref[bh],
                                    preferred_element_type=f32))
            return jnp.concatenate(ctxs, axis=1), ls[0], ls[1]

        def mk(comm, hop, half, sends, recvs, tgt):
            src = 0 if hop == N_DEV - 2 else hop
            return pltpu.make_async_remote_copy(
                src_ref=comm.at[src, half], dst_ref=comm.at[hop + 1, half],
                send_sem=sends.at[hop, half], recv_sem=recvs.at[hop, half],
                device_id=(tgt,), device_id_type=pl.DeviceIdType.MESH)

        accs = [[[None] * (NR + 1) for _ in range(NH)] for _ in range(2)]
        dR = [[None] * NH for _ in range(N_DEV - 1)]
        dL = [[None] * NH for _ in range(N_DEV - 1)]

        def accrows(d, half, slot):
            comm = commR if d == 0 else commL
            for r in range(NR + 1):
                accs[d][half][r] = accs[d][half][r] + comm[slot, half, r]

        for half in range(NH):
            for p in range(NP):
                pair, l0, l1 = partial_half(p, half)
                d, pb = (0, p) if p < NR else (1, p - NR)
                comm = commR if d == 0 else commL
                comm[0, half, pb] = pair.astype(bf16)
                comm[0, half, NR, :, 2 * pb:2 * pb + 1] = l0.astype(bf16)
                comm[0, half, NR, :, 2 * pb + 1:2 * pb + 2] = l1.astype(bf16)
                accs[d][half][pb] = pair.astype(bf16)
                if half == 1 and p == NP - 1:
                    dR[2][0].wait_recv()
                    dL[2][0].wait_recv()
                    accrows(0, 0, 3)
                    accrows(1, 0, 3)
                if half == 1 and p == NP - 2:
                    for dd, comm2, sends, recvs, tgt in (
                            (dR, commR, sendR, recvR, right),
                            (dL, commL, sendL, recvL, left)):
                        dd[0][0].wait_recv()
                        dd[1][0] = mk(comm2, 1, 0, sends, recvs, tgt)
                        dd[1][0].start()
                    accrows(0, 0, 1)
                    accrows(1, 0, 1)
            dR[0][half] = mk(commR, 0, half, sendR, recvR, right)
            dR[0][half].start()
            dL[0][half] = mk(commL, 0, half, sendL, recvL, left)
            dL[0][half].start()
            dR[2][half] = mk(commR, 2, half, sendR, recvR, left)
            dR[2][half].start()
            dL[2][half] = mk(commL, 2, half, sendL, recvL, right)
            dL[2][half].start()
            accs[0][half][NR] = commR[0, half, NR]
            accs[1][half][NR] = commL[0, half, NR]

        def finish(b, half):
            rec = 1.0 / accs[b][half][NR].astype(f32)
            ctx = jnp.concatenate(
                [accs[b][half][pb].astype(f32) for pb in range(NR)], axis=1)
            mul = jnp.concatenate(
                [jnp.broadcast_to(rec[:, c:c + 1], (SH, Dh))
                 for c in range(2 * NR)], axis=1)
            ctxn = (ctx * mul).astype(bf16)
            out_ref[b, pl.ds(half * SH, SH), :] = jnp.dot(
                ctxn, wo_ref[:, :], preferred_element_type=f32)

        dR[1][0].wait_recv()
        dL[1][0].wait_recv()
        accrows(0, 0, 2)
        accrows(1, 0, 2)
        for dd, comm2, sends, recvs, tgt in (
                (dR, commR, sendR, recvR, right),
                (dL, commL, sendL, recvL, left)):
            dd[0][1].wait_recv()
            dd[1][1] = mk(comm2, 1, 1, sends, recvs, tgt)
            dd[1][1].start()
        finish(0, 0)
        finish(1, 0)
        accrows(0, 1, 1)
        accrows(1, 1, 1)
        dR[2][1].wait_recv()
        dL[2][1].wait_recv()
        accrows(0, 1, 3)
        accrows(1, 1, 3)
        dR[1][1].wait_recv()
        dL[1][1].wait_recv()
        accrows(0, 1, 2)
        accrows(1, 1, 2)
        finish(0, 1)
        finish(1, 1)

        for hop in range(N_DEV - 1):
            for half in range(NH):
                dR[hop][half].wait_send()
                dL[hop][half].wait_send()

    return pl.pallas_call(
        body,
        out_shape=jax.ShapeDtypeStruct((B, Sq, E), jnp.float32),
        in_specs=[pl.BlockSpec(memory_space=pltpu.VMEM)] * 5,
        out_specs=pl.BlockSpec(memory_space=pltpu.VMEM),
        scratch_shapes=[
            pltpu.VMEM((N_DEV, NH, NR + 1, SH, 2 * Dh), bf16),
            pltpu.VMEM((N_DEV, NH, NR + 1, SH, 2 * Dh), bf16),
            pltpu.SemaphoreType.DMA((N_DEV - 1, NH)),
            pltpu.SemaphoreType.DMA((N_DEV - 1, NH)),
            pltpu.SemaphoreType.DMA((N_DEV - 1, NH)),
            pltpu.SemaphoreType.DMA((N_DEV - 1, NH)),
        ],
        compiler_params=pltpu.CompilerParams(collective_id=0),
    )(xb, Wqb, Kh, Vh, Wob)
